# Initial kernel scaffold; baseline (speedup 1.0000x reference)
import jax
import jax.numpy as jnp
from jax.experimental import pallas as pl


def kernel(x, edge_index, edge_vector, W):
    raise NotImplementedError("write your pallas kernel here")


# speedup vs baseline: 1.9616x; 1.9616x over previous
import functools
import jax
import jax.numpy as jnp
from jax import lax
from jax.experimental import pallas as pl
from jax.experimental.pallas import tpu as pltpu
from jax.experimental.pallas import tpu_sc as plsc

NUM_RADIAL = 8
RBF_START = 0.0
RBF_STOP = 5.0
WIDTH = (RBF_STOP - RBF_START) / (NUM_RADIAL - 1)

NC = 2
NS = 16
NW = NC * NS
CHUNK = 80


def _make_gather(N, E, D):
    epw = E // NW
    nch = epw // CHUNK
    mesh = plsc.VectorSubcoreMesh(core_axis_name="c", subcore_axis_name="s")

    @functools.partial(
        pl.kernel,
        mesh=mesh,
        out_type=jax.ShapeDtypeStruct((E, D), jnp.float32),
        scratch_types=[
            pltpu.VMEM((nch, CHUNK), jnp.int32),
            pltpu.VMEM((CHUNK, D), jnp.float32),
            pltpu.SemaphoreType.DMA,
        ],
    )
    def gather_k(x_hbm, src_hbm, xj_hbm, idx_all, rows, gsem):
        wid = lax.axis_index("s") * NC + lax.axis_index("c")
        pltpu.sync_copy(src_hbm.at[wid], idx_all)
        wbase = wid * epw

        def body(j, carry):
            pltpu.async_copy(x_hbm.at[idx_all.at[j]], rows, gsem).wait()
            pltpu.sync_copy(rows, xj_hbm.at[pl.ds(wbase + j * CHUNK, CHUNK)])
            return carry

        lax.fori_loop(0, nch, body, 0)

    return gather_k


def _make_scatter(N, E, D, npad):
    epw = E // NW
    nch = epw // CHUNK
    rpt = npad // NS
    mesh = plsc.VectorSubcoreMesh(core_axis_name="c", subcore_axis_name="s")

    @functools.partial(
        pl.kernel,
        mesh=mesh,
        out_type=jax.ShapeDtypeStruct((NC, npad, D), jnp.float32),
        scratch_types=[
            pltpu.VMEM((nch, CHUNK), jnp.int32),
            pltpu.VMEM((CHUNK, D), jnp.float32),
            pltpu.VMEM_SHARED((npad, D), jnp.float32),
        ],
    )
    def scatter_k(msg_hbm, dst_hbm, zeros_hbm, part_hbm, idx_all, msg_v, acc):
        cid = lax.axis_index("c")
        sid = lax.axis_index("s")
        wid = sid * NC + cid
        pltpu.sync_copy(zeros_hbm, acc.at[pl.ds(sid * rpt, rpt)])
        pltpu.sync_copy(dst_hbm.at[wid], idx_all)
        plsc.subcore_barrier()
        wbase = wid * epw

        def body(j, carry):
            pltpu.sync_copy(msg_hbm.at[pl.ds(wbase + j * CHUNK, CHUNK)], msg_v)
            pltpu.sync_copy(msg_v, acc.at[idx_all.at[j]], add=True)
            return carry

        lax.fori_loop(0, nch, body, 0)
        plsc.subcore_barrier()
        pltpu.sync_copy(acc.at[pl.ds(sid * rpt, rpt)],
                        part_hbm.at[cid].at[pl.ds(sid * rpt, rpt)])

    return scatter_k


def _tp_body(ev_ref, xj_ref, wt_ref, out_ref):
    ev = ev_ref[...]
    d2 = jnp.sum(ev * ev, axis=1, keepdims=True)
    d = jnp.sqrt(d2 + 1e-12)
    r = lax.broadcasted_iota(jnp.int32, (1, NUM_RADIAL), 1).astype(jnp.float32)
    centers = RBF_START + r * WIDTH
    scaling = 1.0 / jnp.sqrt(2.0 * jnp.pi)
    rbf = scaling * jnp.exp(-0.5 * ((d - centers) / WIDTH) ** 2)
    xj = xj_ref[...]
    acc = jnp.zeros(out_ref.shape, jnp.float32)
    for rr in range(NUM_RADIAL):
        acc = acc + rbf[:, rr:rr + 1] * jnp.dot(
            xj, wt_ref[rr], preferred_element_type=jnp.float32)
    out_ref[...] = acc


def _tensor_product(xj, ev, wt, block=512):
    E, D = xj.shape
    grid = E // block
    return pl.pallas_call(
        _tp_body,
        grid=(grid,),
        in_specs=[
            pl.BlockSpec((block, 3), lambda i: (i, 0)),
            pl.BlockSpec((block, D), lambda i: (i, 0)),
            pl.BlockSpec(wt.shape, lambda i: (0, 0, 0)),
        ],
        out_specs=pl.BlockSpec((block, D), lambda i: (i, 0)),
        out_shape=jax.ShapeDtypeStruct((E, D), jnp.float32),
    )(ev, xj, wt)


def _sum_body(p_ref, o_ref):
    o_ref[...] = p_ref[0] + p_ref[1]


def _sum_parts(parts, N, block=2000):
    _, _, D = parts.shape
    return pl.pallas_call(
        _sum_body,
        grid=(N // block,),
        in_specs=[pl.BlockSpec((2, block, D), lambda i: (0, i, 0))],
        out_specs=pl.BlockSpec((block, D), lambda i: (i, 0)),
        out_shape=jax.ShapeDtypeStruct((N, D), jnp.float32),
    )(parts)


def kernel(x, edge_index, edge_vector, W):
    N, D = x.shape
    E = edge_index.shape[1]
    nch = E // NW // CHUNK
    src3 = edge_index[0].reshape(NW, nch, CHUNK)
    dst3 = edge_index[1].reshape(NW, nch, CHUNK)
    wt = jnp.transpose(W, (1, 0, 2))
    npad = -(-N // (NS * 8)) * NS * 8
    zeros = jnp.zeros((npad // NS, D), jnp.float32)

    xj = _make_gather(N, E, D)(x, src3)
    msg = _tensor_product(xj, edge_vector, wt)
    parts = _make_scatter(N, E, D, npad)(msg, dst3, zeros)
    return _sum_parts(parts, N)


# speedup vs baseline: 2.0352x; 1.0376x over previous
import functools
import jax
import jax.numpy as jnp
from jax import lax
from jax.experimental import pallas as pl
from jax.experimental.pallas import tpu as pltpu
from jax.experimental.pallas import tpu_sc as plsc

NUM_RADIAL = 8
RBF_START = 0.0
RBF_STOP = 5.0
WIDTH = (RBF_STOP - RBF_START) / (NUM_RADIAL - 1)

NC = 2
NS = 16
NW = NC * NS
CHUNK = 80
CHUNK_S = 40
NBUF = 5
NBUF_S = 2


def _make_gather(N, E, D):
    epw = E // NW
    nch = epw // CHUNK
    mesh = plsc.VectorSubcoreMesh(core_axis_name="c", subcore_axis_name="s")

    @functools.partial(
        pl.kernel,
        mesh=mesh,
        out_type=jax.ShapeDtypeStruct((E, D), jnp.float32),
        scratch_types=[
            pltpu.VMEM((nch, CHUNK), jnp.int32),
            pltpu.VMEM((NBUF, CHUNK, D), jnp.float32),
            pltpu.SemaphoreType.DMA,
            pltpu.SemaphoreType.DMA,
        ],
    )
    def gather_k(x_hbm, src_hbm, xj_hbm, idx_all, rows, gsem, wsem):
        wid = lax.axis_index("s") * NC + lax.axis_index("c")
        pltpu.sync_copy(src_hbm.at[wid], idx_all)
        wbase = wid * epw

        def _fire_gather(j, b):
            pltpu.async_copy(x_hbm.at[idx_all.at[j]], rows.at[b], gsem)

        def _write_slice(j):
            return xj_hbm.at[pl.ds(wbase + j * CHUNK, CHUNK)]

        for g in range(NBUF - 1):
            _fire_gather(g, g)

        def outer(i, carry):
            for b in range(NBUF):
                j = i * NBUF + b
                pltpu.make_async_copy(x_hbm.at[idx_all.at[j]],
                                      rows.at[b], gsem).wait()
                pltpu.async_copy(rows.at[b], _write_slice(j), wsem)
                g = j + NBUF - 1
                bb = (b + NBUF - 1) % NBUF

                @pl.when(jnp.logical_and(g >= NBUF, g < nch))
                def _():
                    pltpu.make_async_copy(rows.at[bb], _write_slice(0),
                                          wsem).wait()
                    _fire_gather(g, bb)

                @pl.when(g == NBUF - 1)
                def _():
                    _fire_gather(NBUF - 1, NBUF - 1)

            return carry

        lax.fori_loop(0, nch // NBUF, outer, 0)
        for _ in range(NBUF):
            pltpu.make_async_copy(rows.at[0], _write_slice(0), wsem).wait()

    return gather_k


def _make_scatter(N, E, D, npad):
    epw = E // NW
    chunk = CHUNK_S
    nch = epw // chunk
    rpt = npad // NS
    mesh = plsc.VectorSubcoreMesh(core_axis_name="c", subcore_axis_name="s")

    @functools.partial(
        pl.kernel,
        mesh=mesh,
        out_type=jax.ShapeDtypeStruct((NC, npad, D), jnp.float32),
        scratch_types=[
            pltpu.VMEM((nch, chunk), jnp.int32),
            pltpu.VMEM((NBUF_S, chunk, D), jnp.float32),
            pltpu.VMEM_SHARED((npad, D), jnp.float32),
            pltpu.SemaphoreType.DMA,
        ],
    )
    def scatter_k(msg_hbm, dst_hbm, zeros_hbm, part_hbm, idx_all, msg_v, acc,
                  lsem):
        cid = lax.axis_index("c")
        sid = lax.axis_index("s")
        wid = sid * NC + cid
        pltpu.sync_copy(zeros_hbm, acc.at[pl.ds(sid * rpt, rpt)])
        pltpu.sync_copy(dst_hbm.at[wid], idx_all)
        plsc.subcore_barrier()
        wbase = wid * epw

        def _load_slice(j):
            return msg_hbm.at[pl.ds(wbase + j * chunk, chunk)]

        for g in range(NBUF_S - 1):
            pltpu.async_copy(_load_slice(g), msg_v.at[g], lsem)

        def outer(i, carry):
            for b in range(NBUF_S):
                j = i * NBUF_S + b
                pltpu.make_async_copy(_load_slice(0), msg_v.at[b],
                                      lsem).wait()
                g = j + NBUF_S - 1
                bb = (b + NBUF_S - 1) % NBUF_S

                @pl.when(g < nch)
                def _():
                    pltpu.async_copy(_load_slice(g), msg_v.at[bb], lsem)

                pltpu.sync_copy(msg_v.at[b], acc.at[idx_all.at[j]], add=True)
            return carry

        lax.fori_loop(0, nch // NBUF_S, outer, 0)
        plsc.subcore_barrier()
        pltpu.sync_copy(acc.at[pl.ds(sid * rpt, rpt)],
                        part_hbm.at[cid].at[pl.ds(sid * rpt, rpt)])

    return scatter_k


def _tp_body(ev_ref, xj_ref, wf_ref, out_ref):
    ev = ev_ref[...]
    d2 = jnp.sum(ev * ev, axis=1, keepdims=True)
    d = jnp.sqrt(d2 + 1e-12)
    r = lax.broadcasted_iota(jnp.int32, (1, NUM_RADIAL), 1).astype(jnp.float32)
    centers = RBF_START + r * WIDTH
    scaling = 1.0 / jnp.sqrt(2.0 * jnp.pi)
    rbf = scaling * jnp.exp(-0.5 * ((d - centers) / WIDTH) ** 2)
    xj = xj_ref[...].astype(jnp.bfloat16)
    acc = jnp.zeros(out_ref.shape, jnp.float32)
    for i in range(NUM_RADIAL):
        acc = acc + rbf[:, i:i + 1] * jnp.dot(
            xj, wf_ref[pl.ds(i * 128, 128)],
            preferred_element_type=jnp.float32)
    out_ref[...] = acc


def _tensor_product(xj, ev, wf, block=512):
    E, D = xj.shape
    grid = E // block
    return pl.pallas_call(
        _tp_body,
        grid=(grid,),
        in_specs=[
            pl.BlockSpec((block, 3), lambda i: (i, 0)),
            pl.BlockSpec((block, D), lambda i: (i, 0)),
            pl.BlockSpec(wf.shape, lambda i: (0, 0)),
        ],
        out_specs=pl.BlockSpec((block, D), lambda i: (i, 0)),
        out_shape=jax.ShapeDtypeStruct((E, D), jnp.float32),
    )(ev, xj, wf)


def _sum_body(p_ref, o_ref):
    o_ref[...] = p_ref[0] + p_ref[1]


def _sum_parts(parts, N, block=2000):
    _, _, D = parts.shape
    return pl.pallas_call(
        _sum_body,
        grid=(N // block,),
        in_specs=[pl.BlockSpec((2, block, D), lambda i: (0, i, 0))],
        out_specs=pl.BlockSpec((block, D), lambda i: (i, 0)),
        out_shape=jax.ShapeDtypeStruct((N, D), jnp.float32),
    )(parts)


def kernel(x, edge_index, edge_vector, W):
    N, D = x.shape
    E = edge_index.shape[1]
    src3 = edge_index[0].reshape(NW, E // NW // CHUNK, CHUNK)
    dst3 = edge_index[1].reshape(NW, E // NW // CHUNK_S, CHUNK_S)
    wf = jnp.transpose(W, (1, 0, 2)).reshape(
        NUM_RADIAL * D, D).astype(jnp.bfloat16)
    npad = -(-N // (NS * 8)) * NS * 8
    zeros = jnp.zeros((npad // NS, D), jnp.float32)

    xj = _make_gather(N, E, D)(x, src3)
    msg = _tensor_product(xj, edge_vector, wf)
    parts = _make_scatter(N, E, D, npad)(msg, dst3, zeros)
    return _sum_parts(parts, N)


# speedup vs baseline: 2.5726x; 1.2640x over previous
import functools
import jax
import jax.numpy as jnp
from jax import lax
from jax.experimental import pallas as pl
from jax.experimental.pallas import tpu as pltpu
from jax.experimental.pallas import tpu_sc as plsc

NUM_RADIAL = 8
RBF_START = 0.0
RBF_STOP = 5.0
WIDTH = (RBF_STOP - RBF_START) / (NUM_RADIAL - 1)

NC = 2
NS = 16
NW = NC * NS
CHUNK = 40
NBUF = 5
NBUF_S = 2


def _make_gather(N, E, D):
    epw = E // NW
    nch = epw // CHUNK
    mesh = plsc.VectorSubcoreMesh(core_axis_name="c", subcore_axis_name="s")

    @functools.partial(
        pl.kernel,
        mesh=mesh,
        out_type=jax.ShapeDtypeStruct((E, D), jnp.float32),
        scratch_types=[
            pltpu.VMEM((nch, CHUNK), jnp.int32),
            pltpu.VMEM((NBUF * CHUNK, D), jnp.float32),
            pltpu.SemaphoreType.DMA,
            pltpu.SemaphoreType.DMA,
        ],
    )
    def gather_k(x_hbm, src_hbm, xj_hbm, idx_all, rows, gsem, wsem):
        wid = lax.axis_index("s") * NC + lax.axis_index("c")
        pltpu.sync_copy(src_hbm.at[wid], idx_all)
        wbase = wid * epw

        def _buf(j):
            return rows.at[pl.ds((j % NBUF) * CHUNK, CHUNK)]

        def _wslice(j):
            return xj_hbm.at[pl.ds(wbase + j * CHUNK, CHUNK)]

        def _fire(j):
            pltpu.async_copy(x_hbm.at[idx_all.at[j]], _buf(j), gsem)

        for g in range(NBUF - 1):
            _fire(g)

        def body(j, carry):
            pltpu.make_async_copy(x_hbm.at[idx_all.at[j]], _buf(j),
                                  gsem).wait()
            pltpu.async_copy(_buf(j), _wslice(j), wsem)
            g = j + NBUF - 1

            @pl.when(jnp.logical_and(g >= NBUF, g < nch))
            def _():
                pltpu.make_async_copy(_buf(0), _wslice(0), wsem).wait()
                _fire(g)

            @pl.when(g == NBUF - 1)
            def _():
                _fire(NBUF - 1)

            return carry

        lax.fori_loop(0, nch, body, 0)
        for _ in range(NBUF):
            pltpu.make_async_copy(_buf(0), _wslice(0), wsem).wait()

    return gather_k


def _make_scatter(N, E, D, npad):
    epw = E // NW
    nch = epw // CHUNK
    rpt = npad // NS
    mesh = plsc.VectorSubcoreMesh(core_axis_name="c", subcore_axis_name="s")

    @functools.partial(
        pl.kernel,
        mesh=mesh,
        out_type=jax.ShapeDtypeStruct((NC, npad, D), jnp.float32),
        scratch_types=[
            pltpu.VMEM((nch, CHUNK), jnp.int32),
            pltpu.VMEM((NBUF_S * CHUNK, D), jnp.float32),
            pltpu.VMEM_SHARED((npad, D), jnp.float32),
            pltpu.SemaphoreType.DMA,
            pltpu.SemaphoreType.DMA,
        ],
    )
    def scatter_k(msg_hbm, dst_hbm, init_hbm, part_hbm, idx_all, msg_v, acc,
                  lsem, asem):
        cid = lax.axis_index("c")
        sid = lax.axis_index("s")
        wid = sid * NC + cid
        rows = pl.ds(sid * rpt, rpt)
        pltpu.sync_copy(init_hbm.at[cid].at[rows], acc.at[rows])
        pltpu.sync_copy(dst_hbm.at[wid], idx_all)
        plsc.subcore_barrier()
        wbase = wid * epw

        def _buf(j):
            return msg_v.at[pl.ds((j % NBUF_S) * CHUNK, CHUNK)]

        def _lslice(j):
            return msg_hbm.at[pl.ds(wbase + j * CHUNK, CHUNK)]

        for g in range(NBUF_S - 1):
            pltpu.async_copy(_lslice(g), _buf(g), lsem)

        def body(j, carry):
            pltpu.make_async_copy(_lslice(0), _buf(j), lsem).wait()
            pltpu.async_copy(_buf(j), acc.at[idx_all.at[j]], asem, add=True)
            g = j + NBUF_S - 1

            @pl.when(jnp.logical_and(g >= NBUF_S, g < nch))
            def _():
                pltpu.make_async_copy(_buf(0), acc.at[idx_all.at[0]],
                                      asem).wait()
                pltpu.async_copy(_lslice(g), _buf(g), lsem)

            @pl.when(g == NBUF_S - 1)
            def _():
                pltpu.async_copy(_lslice(NBUF_S - 1), _buf(NBUF_S - 1), lsem)

            return carry

        lax.fori_loop(0, nch, body, 0)
        for _ in range(NBUF_S):
            pltpu.make_async_copy(_buf(0), acc.at[idx_all.at[0]], asem).wait()
        plsc.subcore_barrier()
        pltpu.sync_copy(acc.at[rows], part_hbm.at[cid].at[rows])

    return scatter_k


def _tp_body(ev_ref, xj_ref, wf_ref, out_ref):
    ev = ev_ref[...]
    d2 = jnp.sum(ev * ev, axis=1, keepdims=True)
    d = jnp.sqrt(d2 + 1e-12)
    r = lax.broadcasted_iota(jnp.int32, (1, NUM_RADIAL), 1).astype(jnp.float32)
    centers = RBF_START + r * WIDTH
    scaling = 1.0 / jnp.sqrt(2.0 * jnp.pi)
    rbf = scaling * jnp.exp(-0.5 * ((d - centers) / WIDTH) ** 2)
    xj = xj_ref[...].astype(jnp.bfloat16)
    acc = jnp.zeros(out_ref.shape, jnp.float32)
    for i in range(NUM_RADIAL):
        acc = acc + rbf[:, i:i + 1] * jnp.dot(
            xj, wf_ref[pl.ds(i * 128, 128)],
            preferred_element_type=jnp.float32)
    out_ref[...] = acc


def _tensor_product(xj, ev, wf, block=640):
    E, D = xj.shape
    grid = E // block
    return pl.pallas_call(
        _tp_body,
        grid=(grid,),
        in_specs=[
            pl.BlockSpec((block, 3), lambda i: (i, 0)),
            pl.BlockSpec((block, D), lambda i: (i, 0)),
            pl.BlockSpec(wf.shape, lambda i: (0, 0)),
        ],
        out_specs=pl.BlockSpec((block, D), lambda i: (i, 0)),
        out_shape=jax.ShapeDtypeStruct((E, D), jnp.float32),
    )(ev, xj, wf)


def _sum_body(p_ref, o_ref):
    o_ref[...] = p_ref[0] + p_ref[1]


def _sum_parts(parts, N, block=2000):
    _, _, D = parts.shape
    return pl.pallas_call(
        _sum_body,
        grid=(N // block,),
        in_specs=[pl.BlockSpec((2, block, D), lambda i: (0, i, 0))],
        out_specs=pl.BlockSpec((block, D), lambda i: (i, 0)),
        out_shape=jax.ShapeDtypeStruct((N, D), jnp.float32),
    )(parts)


def kernel(x, edge_index, edge_vector, W):
    N, D = x.shape
    E = edge_index.shape[1]
    Eh = E // 2
    nch = Eh // NW // CHUNK
    src = edge_index[0]
    dst = edge_index[1]
    wf = jnp.transpose(W, (1, 0, 2)).reshape(
        NUM_RADIAL * D, D).astype(jnp.bfloat16)
    npad = -(-N // (NS * 8)) * NS * 8
    zeros = jnp.zeros((NC, npad, D), jnp.float32)

    gather = _make_gather(N, Eh, D)
    scatter = _make_scatter(N, Eh, D, npad)

    parts = zeros
    msgs = [None, None]
    xjs = [None, None]
    for h in range(2):
        sl = slice(h * Eh, (h + 1) * Eh)
        xjs[h] = gather(x, src[sl].reshape(NW, nch, CHUNK))
    for h in range(2):
        sl = slice(h * Eh, (h + 1) * Eh)
        msgs[h] = _tensor_product(xjs[h], edge_vector[sl], wf)
    for h in range(2):
        sl = slice(h * Eh, (h + 1) * Eh)
        parts = scatter(msgs[h], dst[sl].reshape(NW, nch, CHUNK), parts)
    return _sum_parts(parts, N)
